# baseline (speedup 1.0000x reference)
import jax
import jax.numpy as jnp
from jax.experimental import pallas as pl
from jax.experimental.pallas import tpu as pltpu


def _bitonic_kernel(v_ref, mask_ref, out_ref):
    s = v_ref.shape[2]
    dh = v_ref.shape[3]
    log_s = s.bit_length() - 1

    x = v_ref[0, 0]
    lane_desc = mask_ref[0:1, :] != 0
    iota = jax.lax.broadcasted_iota(jnp.int32, (s, dh), 0)

    def stage(x, j, k):
        t1 = (iota & j) == 0
        t2 = (iota & k) != 0
        partner = jnp.where(t1, pltpu.roll(x, -j, 0), pltpu.roll(x, j, 0))
        lo = jnp.minimum(x, partner)
        hi = jnp.maximum(x, partner)
        keep_min = t1 ^ t2 ^ lane_desc
        return jnp.where(keep_min, lo, hi)

    def outer(kk, x):
        k = jnp.int32(1) << kk

        def inner(t, x):
            j = k >> (t + 1)
            return stage(x, j, k)

        return jax.lax.fori_loop(0, kk, inner, x)

    x = jax.lax.fori_loop(1, log_s + 1, outer, x)
    out_ref[0, 0] = x


def _sort_modulated(v, mask):
    b, h, s, dh = v.shape
    return pl.pallas_call(
        _bitonic_kernel,
        grid=(b, h),
        in_specs=[
            pl.BlockSpec((1, 1, s, dh), lambda i, j: (i, j, 0, 0)),
            pl.BlockSpec((8, dh), lambda i, j: (0, 0)),
        ],
        out_specs=pl.BlockSpec((1, 1, s, dh), lambda i, j: (i, j, 0, 0)),
        out_shape=jax.ShapeDtypeStruct(v.shape, v.dtype),
    )(v, mask)


@jax.jit
def kernel(q, k, v, col_descend):
    del q, k
    dh = v.shape[-1]
    cols = jnp.asarray(col_descend).reshape(-1).astype(jnp.int32)
    mask = jnp.zeros((8, dh), jnp.int32).at[0, cols].set(1)
    out = _sort_modulated(v, mask)
    return (out, out)


# speedup vs baseline: 9.3222x; 9.3222x over previous
import jax
import jax.numpy as jnp
from jax.experimental import pallas as pl
from jax.experimental.pallas import tpu as pltpu

_CH = 256


def _ce_big(x, j, k, desc_extra, lane_desc3):
    ch, dh = x.shape
    m = ch // (2 * j)
    x3 = x.reshape(m, 2 * j, dh)
    a = x3[:, :j]
    b = x3[:, j:]
    lo = jnp.minimum(a, b)
    hi = jnp.maximum(a, b)
    biota = jax.lax.broadcasted_iota(jnp.int32, (m, 1, 1), 0)
    t2 = ((biota * (2 * j)) & k) != 0
    desc = t2 ^ desc_extra ^ lane_desc3
    out_a = jnp.where(desc, hi, lo)
    out_b = jnp.where(desc, lo, hi)
    return jnp.concatenate([out_a, out_b], axis=1).reshape(ch, dh)


def _ce_small(x, j, k, desc_extra, lane_desc, iota_r):
    t1 = (iota_r & j) == 0
    t2 = (iota_r & k) != 0
    partner = jnp.where(t1, pltpu.roll(x, x.shape[0] - j, 0), pltpu.roll(x, j, 0))
    lo = jnp.minimum(x, partner)
    hi = jnp.maximum(x, partner)
    keep_min = t1 ^ t2 ^ desc_extra ^ lane_desc
    return jnp.where(keep_min, lo, hi)


def _bitonic_kernel(v_ref, mask_ref, out_ref):
    s = v_ref.shape[2]
    dh = v_ref.shape[3]
    log_s = s.bit_length() - 1
    ch = _CH
    log_ch = ch.bit_length() - 1
    nch = s // ch

    lane_desc = mask_ref[0:1, :] != 0
    lane_desc3 = lane_desc[None]
    iota_r = jax.lax.broadcasted_iota(jnp.int32, (ch, dh), 0)

    def chunk_stages(x, c, kk_range):
        for kk in kk_range:
            k = 1 << kk
            if k >= ch:
                desc_extra = ((c * ch) & k) != 0
            else:
                desc_extra = False
            for jj in range(min(kk, log_ch) - 1, -1, -1):
                j = 1 << jj
                if j >= 8:
                    x = _ce_big(x, j, k, desc_extra, lane_desc3)
                else:
                    x = _ce_small(x, j, k, desc_extra, lane_desc, iota_r)
        return x

    def phase_a(c, carry):
        x = v_ref[0, 0, pl.ds(c * ch, ch), :]
        x = chunk_stages(x, c, range(1, log_ch + 1))
        out_ref[0, 0, pl.ds(c * ch, ch), :] = x
        return carry

    jax.lax.fori_loop(0, nch, phase_a, 0)

    for kk in range(log_ch + 1, log_s + 1):
        k = 1 << kk

        for jj in range(kk - 1, log_ch - 1, -1):
            j = 1 << jj
            sub_per_pair = j // ch
            n_iter = s // (2 * ch)

            def cross(t, carry, j=j, k=k, sub_per_pair=sub_per_pair):
                p = t // sub_per_pair
                q = t - p * sub_per_pair
                abase = p * (2 * j) + q * ch
                a = out_ref[0, 0, pl.ds(abase, ch), :]
                b = out_ref[0, 0, pl.ds(abase + j, ch), :]
                lo = jnp.minimum(a, b)
                hi = jnp.maximum(a, b)
                d = (((p * (2 * j)) & k) != 0) ^ lane_desc
                out_ref[0, 0, pl.ds(abase, ch), :] = jnp.where(d, hi, lo)
                out_ref[0, 0, pl.ds(abase + j, ch), :] = jnp.where(d, lo, hi)
                return carry

            jax.lax.fori_loop(0, n_iter, cross, 0)

        def local_pass(c, carry, kk=kk):
            x = out_ref[0, 0, pl.ds(c * ch, ch), :]
            x = chunk_stages(x, c, [kk])
            out_ref[0, 0, pl.ds(c * ch, ch), :] = x
            return carry

        jax.lax.fori_loop(0, nch, local_pass, 0)


def _sort_modulated(v, mask):
    b, h, s, dh = v.shape
    return pl.pallas_call(
        _bitonic_kernel,
        grid=(b, h),
        in_specs=[
            pl.BlockSpec((1, 1, s, dh), lambda i, j: (i, j, 0, 0)),
            pl.BlockSpec((8, dh), lambda i, j: (0, 0)),
        ],
        out_specs=pl.BlockSpec((1, 1, s, dh), lambda i, j: (i, j, 0, 0)),
        out_shape=jax.ShapeDtypeStruct(v.shape, v.dtype),
    )(v, mask)


@jax.jit
def kernel(q, k, v, col_descend):
    del q, k
    dh = v.shape[-1]
    cols = jnp.asarray(col_descend).reshape(-1).astype(jnp.int32)
    mask = jnp.zeros((8, dh), jnp.int32).at[0, cols].set(1)
    out = _sort_modulated(v, mask)
    return (out, out)


# speedup vs baseline: 17.9044x; 1.9206x over previous
import jax
import jax.numpy as jnp
from jax.experimental import pallas as pl
from jax.experimental.pallas import tpu as pltpu

_CH = 256


def _ce_big(x, j, k, flip):
    ch, dh = x.shape
    m = ch // (2 * j)
    x3 = x.reshape(m, 2 * j, dh)
    a = x3[:, :j]
    b = x3[:, j:]
    lo = jnp.minimum(a, b)
    hi = jnp.maximum(a, b)
    if k >= ch:
        out_a, out_b = (hi, lo) if flip else (lo, hi)
    else:
        biota = jax.lax.broadcasted_iota(jnp.int32, (m, 1, 1), 0)
        desc = ((biota * (2 * j)) & k) != 0
        if flip:
            desc = ~desc
        out_a = jnp.where(desc, hi, lo)
        out_b = jnp.where(desc, lo, hi)
    return jnp.concatenate([out_a, out_b], axis=1).reshape(ch, dh)


def _ce_small(x, j, k, flip):
    ch, dh = x.shape
    x3 = x.reshape(ch // 8, 8, dh)
    i8 = jax.lax.broadcasted_iota(jnp.int32, (1, 8, 1), 1)
    t1 = (i8 & j) == 0
    if j == 4:
        partner = pltpu.roll(x3, 4, 1)
    else:
        partner = jnp.where(t1, pltpu.roll(x3, 8 - j, 1), pltpu.roll(x3, j, 1))
    lo = jnp.minimum(x3, partner)
    hi = jnp.maximum(x3, partner)
    if k >= ch:
        keep_min = (t1 != flip)
    elif k >= 8:
        biota = jax.lax.broadcasted_iota(jnp.int32, (ch // 8, 1, 1), 0)
        t2 = ((biota * 8) & k) != 0
        keep_min = t1 ^ t2
    else:
        t2 = (i8 & k) != 0
        keep_min = t1 ^ t2
    return jnp.where(keep_min, lo, hi).reshape(ch, dh)


def _ce(x, j, k, flip):
    if j >= 8:
        return _ce_big(x, j, k, flip)
    return _ce_small(x, j, k, flip)


def _bitonic_kernel(v_ref, mask_ref, out_ref):
    s = v_ref.shape[2]
    dh = v_ref.shape[3]
    log_s = s.bit_length() - 1
    ch = _CH
    log_ch = ch.bit_length() - 1
    nch = s // ch

    sign = jnp.where(mask_ref[0:1, :] != 0, jnp.float32(-1.0), jnp.float32(1.0))

    def phase_a(c, parity):
        x = v_ref[0, 0, pl.ds(c * ch, ch), :] * sign
        for kk in range(1, log_ch + 1):
            k = 1 << kk
            flip = parity if k >= ch else False
            for jj in range(kk - 1, -1, -1):
                x = _ce(x, 1 << jj, k, flip)
        out_ref[0, 0, pl.ds(c * ch, ch), :] = x

    def phase_a_even(t, carry):
        phase_a(2 * t, False)
        return carry

    def phase_a_odd(t, carry):
        phase_a(2 * t + 1, True)
        return carry

    jax.lax.fori_loop(0, nch // 2, phase_a_even, 0)
    jax.lax.fori_loop(0, nch // 2, phase_a_odd, 0)

    for kk in range(log_ch + 1, log_s + 1):
        k = 1 << kk
        last_round = k == s

        for jj in range(kk - 1, log_ch - 1, -1):
            j = 1 << jj
            spp = j // ch
            gp = k // (2 * j)
            npairs = s // (2 * j)
            n_asc = npairs if last_round else npairs // 2

            def make_cross(desc, j=j, spp=spp, gp=gp):
                def cross(t, carry):
                    pi = t // spp
                    q = t - pi * spp
                    p = (pi // gp) * (2 * gp) + (pi - (pi // gp) * gp)
                    if desc:
                        p = p + gp
                    abase = p * (2 * j) + q * ch
                    a = out_ref[0, 0, pl.ds(abase, ch), :]
                    b = out_ref[0, 0, pl.ds(abase + j, ch), :]
                    lo = jnp.minimum(a, b)
                    hi = jnp.maximum(a, b)
                    if desc:
                        lo, hi = hi, lo
                    out_ref[0, 0, pl.ds(abase, ch), :] = lo
                    out_ref[0, 0, pl.ds(abase + j, ch), :] = hi
                    return carry
                return cross

            jax.lax.fori_loop(0, n_asc * spp, make_cross(False), 0)
            if n_asc < npairs:
                jax.lax.fori_loop(0, (npairs - n_asc) * spp, make_cross(True), 0)

        g = k // ch
        n_asc_ch = nch if last_round else nch // 2

        def make_local(desc, kk=kk, g=g, unsign=last_round):
            def local_pass(t, carry):
                c = (t // g) * (2 * g) + (t - (t // g) * g)
                if desc:
                    c = c + g
                x = out_ref[0, 0, pl.ds(c * ch, ch), :]
                for jj in range(log_ch - 1, -1, -1):
                    x = _ce(x, 1 << jj, 1 << kk, desc)
                if unsign:
                    x = x * sign
                out_ref[0, 0, pl.ds(c * ch, ch), :] = x
                return carry
            return local_pass

        jax.lax.fori_loop(0, n_asc_ch, make_local(False), 0)
        if n_asc_ch < nch:
            jax.lax.fori_loop(0, nch - n_asc_ch, make_local(True), 0)


def _sort_modulated(v, mask):
    b, h, s, dh = v.shape
    return pl.pallas_call(
        _bitonic_kernel,
        grid=(b, h),
        in_specs=[
            pl.BlockSpec((1, 1, s, dh), lambda i, j: (i, j, 0, 0)),
            pl.BlockSpec((8, dh), lambda i, j: (0, 0)),
        ],
        out_specs=pl.BlockSpec((1, 1, s, dh), lambda i, j: (i, j, 0, 0)),
        out_shape=jax.ShapeDtypeStruct(v.shape, v.dtype),
    )(v, mask)


@jax.jit
def kernel(q, k, v, col_descend):
    del q, k
    dh = v.shape[-1]
    cols = jnp.asarray(col_descend).reshape(-1).astype(jnp.int32)
    mask = jnp.zeros((8, dh), jnp.int32).at[0, cols].set(1)
    out = _sort_modulated(v, mask)
    return (out, out)


# speedup vs baseline: 20.1248x; 1.1240x over previous
import jax
import jax.numpy as jnp
from jax.experimental import pallas as pl
from jax.experimental.pallas import tpu as pltpu

_CH = 256


def _ce_uni_big(y, j, desc):
    n, r, dh = y.shape
    y3 = y.reshape(n * (r // (2 * j)), 2 * j, dh)
    a = y3[:, :j]
    b = y3[:, j:]
    lo = jnp.minimum(a, b)
    hi = jnp.maximum(a, b)
    if desc:
        lo, hi = hi, lo
    return jnp.concatenate([lo, hi], axis=1).reshape(n, r, dh)


def _ce_uni_small(y, j, desc):
    n, r, dh = y.shape
    z = y.reshape(n * (r // 8), 8, dh)
    i8 = jax.lax.broadcasted_iota(jnp.int32, (1, 8, 1), 1)
    t1 = (i8 & j) == 0
    rm = pltpu.roll(z, 8 - j, 1)
    rp = rm if j == 4 else pltpu.roll(z, j, 1)
    if desc:
        out = jnp.where(t1, jnp.maximum(z, rm), jnp.minimum(z, rp))
    else:
        out = jnp.where(t1, jnp.minimum(z, rm), jnp.maximum(z, rp))
    return out.reshape(n, r, dh)


def _ce_uni(y, j, desc):
    return _ce_uni_big(y, j, desc) if j >= 8 else _ce_uni_small(y, j, desc)


def _ce(x, j, k, flip):
    ch, dh = x.shape
    if k >= ch:
        return _ce_uni(x[None], j, flip)[0]
    if k >= 8:
        x4 = x.reshape(ch // (2 * k), 2, k, dh)
        asc = _ce_uni(x4[:, 0], j, flip)
        dsc = _ce_uni(x4[:, 1], j, not flip)
        return jnp.stack([asc, dsc], axis=1).reshape(ch, dh)
    z = x.reshape(ch // 8, 8, dh)
    i8 = jax.lax.broadcasted_iota(jnp.int32, (1, 8, 1), 1)
    t1 = (i8 & j) == 0
    keep_min = t1 ^ ((i8 & k) != 0)
    if flip:
        keep_min = ~keep_min
    partner = jnp.where(t1, pltpu.roll(z, 8 - j, 1), pltpu.roll(z, j, 1))
    lo = jnp.minimum(z, partner)
    hi = jnp.maximum(z, partner)
    return jnp.where(keep_min, lo, hi).reshape(ch, dh)


def _bitonic_kernel(v_ref, mask_ref, out_ref):
    s = v_ref.shape[2]
    dh = v_ref.shape[3]
    log_s = s.bit_length() - 1
    ch = _CH
    log_ch = ch.bit_length() - 1
    nch = s // ch

    sign = jnp.where(mask_ref[0:1, :] != 0, jnp.float32(-1.0), jnp.float32(1.0))

    def phase_a(c, parity):
        x = v_ref[0, 0, pl.ds(c * ch, ch), :] * sign
        for kk in range(1, log_ch + 1):
            k = 1 << kk
            flip = parity if k >= ch else False
            for jj in range(kk - 1, -1, -1):
                x = _ce(x, 1 << jj, k, flip)
        out_ref[0, 0, pl.ds(c * ch, ch), :] = x

    def phase_a_even(t, carry):
        phase_a(2 * t, False)
        return carry

    def phase_a_odd(t, carry):
        phase_a(2 * t + 1, True)
        return carry

    jax.lax.fori_loop(0, nch // 2, phase_a_even, 0)
    jax.lax.fori_loop(0, nch // 2, phase_a_odd, 0)

    for kk in range(log_ch + 1, log_s + 1):
        k = 1 << kk
        last_round = k == s

        for jj in range(kk - 1, log_ch - 1, -1):
            j = 1 << jj
            spp = j // ch
            gp = k // (2 * j)
            npairs = s // (2 * j)
            n_asc = npairs if last_round else npairs // 2

            def make_cross(desc, j=j, spp=spp, gp=gp):
                def cross(t, carry):
                    pi = t // spp
                    q = t - pi * spp
                    p = (pi // gp) * (2 * gp) + (pi - (pi // gp) * gp)
                    if desc:
                        p = p + gp
                    abase = p * (2 * j) + q * ch
                    a = out_ref[0, 0, pl.ds(abase, ch), :]
                    b = out_ref[0, 0, pl.ds(abase + j, ch), :]
                    lo = jnp.minimum(a, b)
                    hi = jnp.maximum(a, b)
                    if desc:
                        lo, hi = hi, lo
                    out_ref[0, 0, pl.ds(abase, ch), :] = lo
                    out_ref[0, 0, pl.ds(abase + j, ch), :] = hi
                    return carry
                return cross

            jax.lax.fori_loop(0, n_asc * spp, make_cross(False), 0)
            if n_asc < npairs:
                jax.lax.fori_loop(0, (npairs - n_asc) * spp, make_cross(True), 0)

        g = k // ch
        n_asc_ch = nch if last_round else nch // 2

        def make_local(desc, kk=kk, g=g, unsign=last_round):
            def local_pass(t, carry):
                c = (t // g) * (2 * g) + (t - (t // g) * g)
                if desc:
                    c = c + g
                x = out_ref[0, 0, pl.ds(c * ch, ch), :]
                for jj in range(log_ch - 1, -1, -1):
                    x = _ce(x, 1 << jj, 1 << kk, desc)
                if unsign:
                    x = x * sign
                out_ref[0, 0, pl.ds(c * ch, ch), :] = x
                return carry
            return local_pass

        jax.lax.fori_loop(0, n_asc_ch, make_local(False), 0)
        if n_asc_ch < nch:
            jax.lax.fori_loop(0, nch - n_asc_ch, make_local(True), 0)


def _sort_modulated(v, mask):
    b, h, s, dh = v.shape
    return pl.pallas_call(
        _bitonic_kernel,
        grid=(b, h),
        in_specs=[
            pl.BlockSpec((1, 1, s, dh), lambda i, j: (i, j, 0, 0)),
            pl.BlockSpec((8, dh), lambda i, j: (0, 0)),
        ],
        out_specs=pl.BlockSpec((1, 1, s, dh), lambda i, j: (i, j, 0, 0)),
        out_shape=jax.ShapeDtypeStruct(v.shape, v.dtype),
    )(v, mask)


@jax.jit
def kernel(q, k, v, col_descend):
    del q, k
    dh = v.shape[-1]
    cols = jnp.asarray(col_descend).reshape(-1).astype(jnp.int32)
    mask = jnp.zeros((8, dh), jnp.int32).at[0, cols].set(1)
    out = _sort_modulated(v, mask)
    return (out, out)
